# Initial kernel scaffold; baseline (speedup 1.0000x reference)
import functools

import jax
import jax.numpy as jnp
from jax import lax
from jax.experimental import pallas as pl
from jax.experimental.pallas import tpu as pltpu
from jax.experimental.pallas import tpu_sc as plsc

D = 64
LANES = 16
NCHUNK = D // LANES
NC, NS = 2, 16
NW = NC * NS
GRP = 8


def _pooler(tok_hbm, len_hbm, table_hbm, out_hbm, len_v, idx_v, rows_v, out_v, sem):
    B = len_hbm.shape[0]
    half = tok_hbm.shape[2]
    n_blocks = B // (NW * GRP)

    c = lax.axis_index("c")
    s = lax.axis_index("s")
    w = s * NC + c

    pltpu.sync_copy(len_hbm, len_v)

    def block_body(i, _):
        blk = i * NW + w
        base = blk * GRP
        pltpu.sync_copy(tok_hbm.at[pl.ds(base, GRP)], idx_v)

        def seq_body(j, _):
            ln = len_v[base + j]
            n0 = jnp.minimum(ln, half)
            n1 = jnp.maximum(ln - half, 0)
            pltpu.async_copy(table_hbm.at[idx_v.at[j, 0]], rows_v.at[0], sem).wait()

            @pl.when(ln > half)
            def _():
                pltpu.async_copy(
                    table_hbm.at[idx_v.at[j, 1]], rows_v.at[1], sem
                ).wait()

            def t_body(h):
                def body(t, acc):
                    return tuple(
                        acc[k] + rows_v[h, t, pl.ds(LANES * k, LANES)]
                        for k in range(NCHUNK)
                    )
                return body

            acc = tuple(jnp.zeros((LANES,), jnp.float32) for _ in range(NCHUNK))
            acc = lax.fori_loop(0, n0, t_body(0), acc)
            acc = lax.fori_loop(0, n1, t_body(1), acc)
            inv = 1.0 / ln.astype(jnp.float32)
            for k in range(NCHUNK):
                out_v[j, pl.ds(LANES * k, LANES)] = acc[k] * inv
            return 0

        lax.fori_loop(0, GRP, seq_body, 0)
        pltpu.sync_copy(out_v, out_hbm.at[pl.ds(base, GRP)])
        return 0

    lax.fori_loop(0, n_blocks, block_body, 0)


@jax.jit
def _run(tok3, lengths, table):
    B = tok3.shape[0]
    half = tok3.shape[2]
    mesh = plsc.VectorSubcoreMesh(core_axis_name="c", subcore_axis_name="s")
    return pl.kernel(
        _pooler,
        out_type=jax.ShapeDtypeStruct((B, D), jnp.float32),
        mesh=mesh,
        scratch_types=[
            pltpu.VMEM((B,), jnp.int32),
            pltpu.VMEM((GRP, 2, half), jnp.int32),
            pltpu.VMEM((2, half, D), jnp.float32),
            pltpu.VMEM((GRP, D), jnp.float32),
            pltpu.SemaphoreType.DMA,
        ],
    )(tok3, lengths, table)


def kernel(token_ids, lengths, table):
    B, L = token_ids.shape
    half = L // 2
    tok3 = token_ids.astype(jnp.int32).reshape(B, 2, half)
    return _run(tok3, lengths.astype(jnp.int32), table)


# speedup vs baseline: 19.7385x; 19.7385x over previous
import functools

import jax
import jax.numpy as jnp
from jax import lax
from jax.experimental import pallas as pl
from jax.experimental.pallas import tpu as pltpu
from jax.experimental.pallas import tpu_sc as plsc

D = 64
LANES = 16
NCHUNK = D // LANES
NC, NS = 2, 16
NW = NC * NS
GRP = 8


def _pooler(tok_hbm, len_hbm, table_hbm, out_hbm, len_v, idx_v, rows_v, out_v, sem):
    B = len_hbm.shape[0]
    half = tok_hbm.shape[2]
    n_blocks = B // (NW * GRP)

    c = lax.axis_index("c")
    s = lax.axis_index("s")
    w = s * NC + c

    pltpu.sync_copy(len_hbm, len_v.at[pl.ds(0, B)])

    def block_body(i, _):
        blk = i * NW + w
        base = blk * GRP
        pltpu.sync_copy(tok_hbm.at[pl.ds(base, GRP)], idx_v)
        lvec = len_v[pl.ds(base, LANES)]
        invv = 1.0 / lvec.astype(jnp.float32)

        for j in range(GRP):
            ln = lvec[j]
            n0 = jnp.minimum(ln, half)
            n1 = jnp.maximum(ln - half, 0)
            pltpu.async_copy(table_hbm.at[idx_v.at[j, 0]], rows_v.at[0], sem).wait()

            @pl.when(ln > half)
            def _():
                pltpu.async_copy(
                    table_hbm.at[idx_v.at[j, 1]], rows_v.at[1], sem
                ).wait()

            def t_body(h):
                def body(t, acc):
                    return tuple(
                        acc[k] + rows_v[h, t, pl.ds(LANES * k, LANES)]
                        for k in range(NCHUNK)
                    )
                return body

            acc = tuple(jnp.zeros((LANES,), jnp.float32) for _ in range(NCHUNK))
            acc = lax.fori_loop(0, n0, t_body(0), acc)
            acc = lax.fori_loop(0, n1, t_body(1), acc)
            inv = invv[j]
            for k in range(NCHUNK):
                out_v[j, pl.ds(LANES * k, LANES)] = acc[k] * inv

        pltpu.sync_copy(out_v, out_hbm.at[pl.ds(base, GRP)])
        return 0

    lax.fori_loop(0, n_blocks, block_body, 0)


@jax.jit
def _run(tok3, lengths, table):
    B = tok3.shape[0]
    half = tok3.shape[2]
    mesh = plsc.VectorSubcoreMesh(core_axis_name="c", subcore_axis_name="s")
    return pl.kernel(
        _pooler,
        out_type=jax.ShapeDtypeStruct((B, D), jnp.float32),
        mesh=mesh,
        scratch_types=[
            pltpu.VMEM((B + LANES,), jnp.int32),
            pltpu.VMEM((GRP, 2, half), jnp.int32),
            pltpu.VMEM((2, half, D), jnp.float32),
            pltpu.VMEM((GRP, D), jnp.float32),
            pltpu.SemaphoreType.DMA,
        ],
        compiler_params=pltpu.CompilerParams(use_tc_tiling_on_sc=False),
    )(tok3, lengths, table)


def kernel(token_ids, lengths, table):
    B, L = token_ids.shape
    half = L // 2
    tok3 = token_ids.astype(jnp.int32).reshape(B, 2, half)
    return _run(tok3, lengths.astype(jnp.int32), table)


# speedup vs baseline: 24.1190x; 1.2219x over previous
import functools

import jax
import jax.numpy as jnp
from jax import lax
from jax.experimental import pallas as pl
from jax.experimental.pallas import tpu as pltpu
from jax.experimental.pallas import tpu_sc as plsc

D = 64
LANES = 16
NCHUNK = D // LANES
NC, NS = 2, 16
NW = NC * NS
G = 4


def _pooler(
    tok_hbm, len_hbm, table_hbm, out_hbm,
    len_v, idx0, idx1, rows0, rows1, out_v,
    sem_r0, sem_r1, sem_i0, sem_i1,
):
    B = len_hbm.shape[0]
    half = tok_hbm.shape[2]
    ngrp = B // (NW * G)

    c = lax.axis_index("c")
    s = lax.axis_index("s")
    w = s * NC + c

    idx_bufs = (idx0, idx1)
    rows_bufs = (rows0, rows1)
    sem_r = (sem_r0, sem_r1)
    sem_i = (sem_i0, sem_i1)

    pltpu.sync_copy(len_hbm, len_v.at[pl.ds(0, B)])

    def gbase(g):
        return (g * NW + w) * G

    def lvec_of(g):
        return len_v[pl.ds(gbase(g), LANES)]

    def rows_copy(j, h, par):
        return pltpu.make_async_copy(
            table_hbm.at[idx_bufs[par].at[j, h]],
            rows_bufs[par].at[j, h],
            sem_r[par],
        )

    def fire_rows(g, par):
        lv = lvec_of(g)
        for j in range(G):
            rows_copy(j, 0, par).start()

            @pl.when(lv[j] > half)
            def _():
                rows_copy(j, 1, par).start()

    def wait_rows(g, par):
        lv = lvec_of(g)
        for j in range(G):
            rows_copy(j, 0, par).wait()

            @pl.when(lv[j] > half)
            def _():
                rows_copy(j, 1, par).wait()

    def idx_copy(g, par):
        return pltpu.make_async_copy(
            tok_hbm.at[pl.ds(gbase(g), G)], idx_bufs[par], sem_i[par]
        )

    def accum_half(rows, j, h, n, acc):
        def body4(q, acc):
            t = q * 4
            for dt in range(4):
                acc = tuple(
                    acc[k] + rows[j, h, t + dt, pl.ds(LANES * k, LANES)]
                    for k in range(NCHUNK)
                )
            return acc

        nq = n // 4
        acc = lax.fori_loop(0, nq, body4, acc)

        def body1(t, acc):
            return tuple(
                acc[k] + rows[j, h, t, pl.ds(LANES * k, LANES)]
                for k in range(NCHUNK)
            )

        return lax.fori_loop(nq * 4, n, body1, acc)

    pltpu.sync_copy(tok_hbm.at[pl.ds(gbase(0), G)], idx0)
    fire_rows(0, 0)
    idx_copy(1, 1).start()

    def gloop(ii, _):
        for par in range(2):
            g = 2 * ii + par
            wait_rows(g, par)

            @pl.when(g + 2 < ngrp)
            def _():
                idx_copy(g + 2, par).start()

            @pl.when(g + 1 < ngrp)
            def _():
                idx_copy(g + 1, 1 - par).wait()
                fire_rows(g + 1, 1 - par)

            lv = lvec_of(g)
            invv = 1.0 / lv.astype(jnp.float32)
            rows = rows_bufs[par]
            for j in range(G):
                ln = lv[j]
                n0 = jnp.minimum(ln, half)
                n1 = ln - n0
                acc = tuple(jnp.zeros((LANES,), jnp.float32) for _ in range(NCHUNK))
                acc = accum_half(rows, j, 0, n0, acc)
                acc = accum_half(rows, j, 1, n1, acc)
                for k in range(NCHUNK):
                    out_v[j, pl.ds(LANES * k, LANES)] = acc[k] * invv[j]
            pltpu.sync_copy(out_v, out_hbm.at[pl.ds(gbase(g), G)])
        return 0

    lax.fori_loop(0, ngrp // 2, gloop, 0)


@jax.jit
def _run(tok3, lengths, table):
    B = tok3.shape[0]
    half = tok3.shape[2]
    mesh = plsc.VectorSubcoreMesh(core_axis_name="c", subcore_axis_name="s")
    return pl.kernel(
        _pooler,
        out_type=jax.ShapeDtypeStruct((B, D), jnp.float32),
        mesh=mesh,
        scratch_types=[
            pltpu.VMEM((B + LANES,), jnp.int32),
            pltpu.VMEM((G, 2, half), jnp.int32),
            pltpu.VMEM((G, 2, half), jnp.int32),
            pltpu.VMEM((G, 2, half, D), jnp.float32),
            pltpu.VMEM((G, 2, half, D), jnp.float32),
            pltpu.VMEM((G, D), jnp.float32),
            pltpu.SemaphoreType.DMA,
            pltpu.SemaphoreType.DMA,
            pltpu.SemaphoreType.DMA,
            pltpu.SemaphoreType.DMA,
        ],
        compiler_params=pltpu.CompilerParams(use_tc_tiling_on_sc=False),
    )(tok3, lengths, table)


def kernel(token_ids, lengths, table):
    B, L = token_ids.shape
    half = L // 2
    tok3 = token_ids.astype(jnp.int32).reshape(B, 2, half)
    return _run(tok3, lengths.astype(jnp.int32), table)


# speedup vs baseline: 24.6083x; 1.0203x over previous
import functools

import jax
import jax.numpy as jnp
from jax import lax
from jax.experimental import pallas as pl
from jax.experimental.pallas import tpu as pltpu
from jax.experimental.pallas import tpu_sc as plsc

D = 64
LANES = 16
NCHUNK = D // LANES
NC, NS = 2, 16
NW = NC * NS
G = 4
C = 40


def _pooler(
    tok_hbm, len_hbm, table_hbm, out_hbm,
    len_v, idx0, idx1, rows0, rows1, out_v,
    sem_r0, sem_r1, sem_i0, sem_i1, sem_o0, sem_o1,
):
    B = len_hbm.shape[0]
    L = tok_hbm.shape[1]
    nch = L // C
    ngrp = B // (NW * G)

    c = lax.axis_index("c")
    s = lax.axis_index("s")
    w = s * NC + c

    idx_bufs = (idx0, idx1)
    rows_bufs = (rows0, rows1)
    sem_r = (sem_r0, sem_r1)
    sem_i = (sem_i0, sem_i1)
    sem_o = (sem_o0, sem_o1)

    pltpu.sync_copy(len_hbm, len_v.at[pl.ds(0, B)])

    def gbase(g):
        return (g * NW + w) * G

    def lvec_of(g):
        return len_v[pl.ds(gbase(g), LANES)]

    def rows_copy(j, h, par):
        return pltpu.make_async_copy(
            table_hbm.at[idx_bufs[par].at[j, pl.ds(h * C, C)]],
            rows_bufs[par].at[j, pl.ds(h * C, C)],
            sem_r[par],
        )

    def fire_rows(g, par):
        lv = lvec_of(g)
        for j in range(G):
            rows_copy(j, 0, par).start()
            for h in range(1, nch):
                @pl.when(lv[j] > h * C)
                def _():
                    rows_copy(j, h, par).start()

    def wait_rows(g, par):
        lv = lvec_of(g)
        for j in range(G):
            rows_copy(j, 0, par).wait()
            for h in range(1, nch):
                @pl.when(lv[j] > h * C)
                def _():
                    rows_copy(j, h, par).wait()

    def idx_copy(g, par):
        return pltpu.make_async_copy(
            tok_hbm.at[pl.ds(gbase(g), G)], idx_bufs[par], sem_i[par]
        )

    def out_copy(g, par):
        return pltpu.make_async_copy(
            out_v.at[par], out_hbm.at[pl.ds(gbase(g), G)], sem_o[par]
        )

    def accum_seq(rows, j, n, acc):
        def body4(q, acc):
            t = q * 4
            for dt in range(4):
                acc = tuple(
                    acc[k] + rows[j, t + dt, pl.ds(LANES * k, LANES)]
                    for k in range(NCHUNK)
                )
            return acc

        nq = n // 4
        acc = lax.fori_loop(0, nq, body4, acc)

        def body1(t, acc):
            return tuple(
                acc[k] + rows[j, t, pl.ds(LANES * k, LANES)]
                for k in range(NCHUNK)
            )

        return lax.fori_loop(nq * 4, n, body1, acc)

    pltpu.sync_copy(tok_hbm.at[pl.ds(gbase(0), G)], idx0)
    fire_rows(0, 0)
    idx_copy(1, 1).start()

    def gloop(ii, _):
        for par in range(2):
            g = 2 * ii + par
            wait_rows(g, par)

            @pl.when(g + 2 < ngrp)
            def _():
                idx_copy(g + 2, par).start()

            @pl.when(g + 1 < ngrp)
            def _():
                idx_copy(g + 1, 1 - par).wait()
                fire_rows(g + 1, 1 - par)

            @pl.when(g >= 2)
            def _():
                out_copy(g - 2, par).wait()

            lv = lvec_of(g)
            invv = 1.0 / lv.astype(jnp.float32)
            rows = rows_bufs[par]
            for j in range(G):
                acc = tuple(jnp.zeros((LANES,), jnp.float32) for _ in range(NCHUNK))
                acc = accum_seq(rows, j, lv[j], acc)
                for k in range(NCHUNK):
                    out_v[par, j, pl.ds(LANES * k, LANES)] = acc[k] * invv[j]
            out_copy(g, par).start()
        return 0

    lax.fori_loop(0, ngrp // 2, gloop, 0)
    out_copy(ngrp - 2, 0).wait()
    out_copy(ngrp - 1, 1).wait()


@jax.jit
def _run(tok, lengths, table):
    B, L = tok.shape
    mesh = plsc.VectorSubcoreMesh(core_axis_name="c", subcore_axis_name="s")
    return pl.kernel(
        _pooler,
        out_type=jax.ShapeDtypeStruct((B, D), jnp.float32),
        mesh=mesh,
        scratch_types=[
            pltpu.VMEM((B + LANES,), jnp.int32),
            pltpu.VMEM((G, L), jnp.int32),
            pltpu.VMEM((G, L), jnp.int32),
            pltpu.VMEM((G, L, D), jnp.float32),
            pltpu.VMEM((G, L, D), jnp.float32),
            pltpu.VMEM((2, G, D), jnp.float32),
            pltpu.SemaphoreType.DMA,
            pltpu.SemaphoreType.DMA,
            pltpu.SemaphoreType.DMA,
            pltpu.SemaphoreType.DMA,
            pltpu.SemaphoreType.DMA,
            pltpu.SemaphoreType.DMA,
        ],
        compiler_params=pltpu.CompilerParams(use_tc_tiling_on_sc=False),
    )(tok, lengths, table)


def kernel(token_ids, lengths, table):
    return _run(token_ids.astype(jnp.int32), lengths.astype(jnp.int32), table)


# speedup vs baseline: 25.0380x; 1.0175x over previous
import functools

import jax
import jax.numpy as jnp
from jax import lax
from jax.experimental import pallas as pl
from jax.experimental.pallas import tpu as pltpu
from jax.experimental.pallas import tpu_sc as plsc

D = 64
LANES = 16
NCHUNK = D // LANES
NC, NS = 2, 16
NW = NC * NS
G = 4
C = 40


def _pooler(
    tok_hbm, len_hbm, table_hbm, out_hbm,
    len_v, idx0, idx1, rows0, rows1, out_v,
    sem_r0, sem_r1, sem_i0, sem_i1, sem_o0, sem_o1,
):
    B = len_hbm.shape[0]
    L = tok_hbm.shape[1]
    nch = L // C
    ngrp = B // (NW * G)

    c = lax.axis_index("c")
    s = lax.axis_index("s")
    w = s * NC + c

    idx_bufs = (idx0, idx1)
    rows_bufs = (rows0, rows1)
    sem_r = (sem_r0, sem_r1)
    sem_i = (sem_i0, sem_i1)
    sem_o = (sem_o0, sem_o1)

    pltpu.sync_copy(len_hbm, len_v.at[pl.ds(0, B)])

    def gbase(g):
        return (g * NW + w) * G

    def lvec_of(g):
        return len_v[pl.ds(gbase(g), LANES)]

    def rows_copy(j, h, par):
        return pltpu.make_async_copy(
            table_hbm.at[idx_bufs[par].at[j, pl.ds(h * C, C)]],
            rows_bufs[par].at[j, pl.ds(h * C, C)],
            sem_r[par],
        )

    def fire_rows(g, par):
        lv = lvec_of(g)
        for j in range(G):
            rows_copy(j, 0, par).start()
            for h in range(1, nch):
                @pl.when(lv[j] > h * C)
                def _():
                    rows_copy(j, h, par).start()

    def wait_rows(g, par):
        lv = lvec_of(g)
        for j in range(G):
            rows_copy(j, 0, par).wait()
            for h in range(1, nch):
                @pl.when(lv[j] > h * C)
                def _():
                    rows_copy(j, h, par).wait()

    def idx_copy(g, par):
        return pltpu.make_async_copy(
            tok_hbm.at[pl.ds(gbase(g), G)], idx_bufs[par], sem_i[par]
        )

    def out_copy(g, par):
        return pltpu.make_async_copy(
            out_v.at[par], out_hbm.at[pl.ds(gbase(g), G)], sem_o[par]
        )

    def accum_seq(rows, j, n, acc):
        def body4(q, acc):
            t = q * 4
            for dt in range(4):
                acc = tuple(
                    acc[k] + rows[j, t + dt, pl.ds(LANES * k, LANES)]
                    for k in range(NCHUNK)
                )
            return acc

        nq = n // 4
        acc = lax.fori_loop(0, nq, body4, acc)

        def body1(t, acc):
            return tuple(
                acc[k] + rows[j, t, pl.ds(LANES * k, LANES)]
                for k in range(NCHUNK)
            )

        return lax.fori_loop(nq * 4, n, body1, acc)

    pltpu.sync_copy(tok_hbm.at[pl.ds(gbase(0), G)], idx0)
    fire_rows(0, 0)
    idx_copy(1, 1).start()

    def gloop(ii, _):
        for par in range(2):
            g = 2 * ii + par

            @pl.when(g + 1 < ngrp)
            def _():
                idx_copy(g + 1, 1 - par).wait()
                fire_rows(g + 1, 1 - par)

            wait_rows(g, par)

            @pl.when(g + 2 < ngrp)
            def _():
                idx_copy(g + 2, par).start()

            @pl.when(g >= 2)
            def _():
                out_copy(g - 2, par).wait()

            lv = lvec_of(g)
            invv = 1.0 / lv.astype(jnp.float32)
            rows = rows_bufs[par]
            for j in range(G):
                acc = tuple(jnp.zeros((LANES,), jnp.float32) for _ in range(NCHUNK))
                acc = accum_seq(rows, j, lv[j], acc)
                for k in range(NCHUNK):
                    out_v[par, j, pl.ds(LANES * k, LANES)] = acc[k] * invv[j]
            out_copy(g, par).start()
        return 0

    lax.fori_loop(0, ngrp // 2, gloop, 0)
    out_copy(ngrp - 2, 0).wait()
    out_copy(ngrp - 1, 1).wait()


@jax.jit
def _run(tok, lengths, table):
    B, L = tok.shape
    mesh = plsc.VectorSubcoreMesh(core_axis_name="c", subcore_axis_name="s")
    return pl.kernel(
        _pooler,
        out_type=jax.ShapeDtypeStruct((B, D), jnp.float32),
        mesh=mesh,
        scratch_types=[
            pltpu.VMEM((B + LANES,), jnp.int32),
            pltpu.VMEM((G, L), jnp.int32),
            pltpu.VMEM((G, L), jnp.int32),
            pltpu.VMEM((G, L, D), jnp.float32),
            pltpu.VMEM((G, L, D), jnp.float32),
            pltpu.VMEM((2, G, D), jnp.float32),
            pltpu.SemaphoreType.DMA,
            pltpu.SemaphoreType.DMA,
            pltpu.SemaphoreType.DMA,
            pltpu.SemaphoreType.DMA,
            pltpu.SemaphoreType.DMA,
            pltpu.SemaphoreType.DMA,
        ],
        compiler_params=pltpu.CompilerParams(use_tc_tiling_on_sc=False),
    )(tok, lengths, table)


def kernel(token_ids, lengths, table):
    return _run(token_ids.astype(jnp.int32), lengths.astype(jnp.int32), table)
